# baseline (speedup 1.0000x reference)
import functools

import jax
import jax.numpy as jnp
from jax import lax
from jax.experimental import pallas as pl
from jax.experimental.pallas import tpu as pltpu
from jax.experimental.pallas import tpu_sc as plsc

N = 10000
E = 320000
D = 128
DOUT = 64
NPAD = 10240
NCORES = 2
NSUB = 16
NTILES = NCORES * NSUB
NBLK = 80
BLKE = 128
MCH = 5
MCB = 16
EPAD = NTILES * NBLK * BLKE
RPT = NPAD // NSUB

_RB = 1024
_GRID = NPAD // _RB

_mesh = plsc.VectorSubcoreMesh(core_axis_name="c", subcore_axis_name="s",
                               num_cores=NCORES, num_subcores=NSUB)


@functools.partial(
    pl.kernel,
    out_type=jax.ShapeDtypeStruct((NCORES, NPAD, D), jnp.float32),
    mesh=_mesh,
    scratch_types=[
        pltpu.VMEM((NBLK, BLKE), jnp.int32),
        pltpu.VMEM((BLKE, D), jnp.float32),
        pltpu.VMEM_SHARED((NPAD, D), jnp.float32),
    ],
)
def _deg_kernel(dst_hbm, out_hbm, dst_v, ones_v, acc_sh):
    c = lax.axis_index("c")
    s = lax.axis_index("s")
    wid = s * NCORES + c
    pltpu.sync_copy(dst_hbm.at[wid], dst_v)
    zrow = jnp.zeros((16,), jnp.float32)
    for r in range(BLKE):
        for q in range(D // 16):
            ones_v[r, pl.ds(q * 16, 16)] = zrow
    for k in range(RPT // BLKE):
        pltpu.sync_copy(ones_v, acc_sh.at[pl.ds(s * RPT + k * BLKE, BLKE)])
    orow = jnp.ones((16,), jnp.float32)
    for r in range(BLKE):
        for q in range(D // 16):
            ones_v[r, pl.ds(q * 16, 16)] = orow
    plsc.subcore_barrier()

    def blk(j, carry):
        pltpu.sync_copy(ones_v, acc_sh.at[dst_v.at[j]], add=True)
        return carry

    lax.fori_loop(0, NBLK, blk, 0)
    plsc.subcore_barrier()
    pltpu.sync_copy(acc_sh.at[pl.ds(s * RPT, RPT)],
                    out_hbm.at[c, pl.ds(s * RPT, RPT)])


@functools.partial(
    pl.kernel,
    out_type=jax.ShapeDtypeStruct((NCORES, NPAD, D), jnp.float32),
    mesh=_mesh,
    scratch_types=[
        pltpu.VMEM((MCB, BLKE), jnp.int32),
        pltpu.VMEM((MCB, BLKE), jnp.int32),
        pltpu.VMEM((BLKE, D), jnp.float32),
        pltpu.VMEM((BLKE, D), jnp.float32),
        pltpu.VMEM_SHARED((NPAD, D), jnp.float32),
        pltpu.SemaphoreType.DMA,
        pltpu.SemaphoreType.DMA,
        pltpu.SemaphoreType.DMA,
        pltpu.SemaphoreType.DMA,
    ],
)
def _msg_kernel(v_hbm, src_hbm, dst_hbm, out_hbm,
                src_v, dst_v, rows_a, rows_b, acc_sh,
                sem_ga, sem_gb, sem_sa, sem_sb):
    c = lax.axis_index("c")
    s = lax.axis_index("s")
    wid = s * NCORES + c
    zrow = jnp.zeros((16,), jnp.float32)
    for r in range(BLKE):
        for q in range(D // 16):
            rows_a[r, pl.ds(q * 16, 16)] = zrow
    for k in range(RPT // BLKE):
        pltpu.sync_copy(rows_a, acc_sh.at[pl.ds(s * RPT + k * BLKE, BLKE)])
    plsc.subcore_barrier()

    for ch in range(MCH):
        pltpu.sync_copy(src_hbm.at[wid, pl.ds(ch * MCB, MCB)], src_v)
        pltpu.sync_copy(dst_hbm.at[wid, pl.ds(ch * MCB, MCB)], dst_v)
        pltpu.async_copy(v_hbm.at[src_v.at[0]], rows_a, sem_ga)
        pltpu.async_copy(v_hbm.at[src_v.at[1]], rows_b, sem_gb)

        def blk(i, carry):
            j = 2 * i
            pltpu.make_async_copy(v_hbm.at[src_v.at[j]], rows_a, sem_ga).wait()
            pltpu.async_copy(rows_a, acc_sh.at[dst_v.at[j]], sem_sa, add=True)
            pltpu.make_async_copy(v_hbm.at[src_v.at[j + 1]], rows_b, sem_gb).wait()
            pltpu.async_copy(rows_b, acc_sh.at[dst_v.at[j + 1]], sem_sb, add=True)
            pltpu.make_async_copy(rows_a, acc_sh.at[dst_v.at[j]], sem_sa).wait()
            pltpu.async_copy(v_hbm.at[src_v.at[(j + 2) % MCB]], rows_a, sem_ga)
            pltpu.make_async_copy(rows_b, acc_sh.at[dst_v.at[j + 1]], sem_sb).wait()
            pltpu.async_copy(v_hbm.at[src_v.at[(j + 3) % MCB]], rows_b, sem_gb)
            return carry

        lax.fori_loop(0, MCB // 2, blk, 0)
        pltpu.make_async_copy(v_hbm.at[src_v.at[0]], rows_a, sem_ga).wait()
        pltpu.make_async_copy(v_hbm.at[src_v.at[1]], rows_b, sem_gb).wait()
    plsc.subcore_barrier()
    pltpu.sync_copy(acc_sh.at[pl.ds(s * RPT, RPT)],
                    out_hbm.at[c, pl.ds(s * RPT, RPT)])


def _dinv(deg_ref):
    deg = (deg_ref[0] + deg_ref[1])[:, 0:1]
    return jnp.where(deg > 0.0, 1.0 / jnp.sqrt(jnp.maximum(deg, 1.0)), 0.0)


def _tc_in_body(x_ref, w_ref, deg_ref, o_ref):
    o_ref[...] = jnp.dot(x_ref[...], w_ref[...],
                         preferred_element_type=jnp.float32,
                         precision=lax.Precision.HIGHEST) * _dinv(deg_ref)


_tc_in = pl.pallas_call(
    _tc_in_body,
    grid=(_GRID,),
    in_specs=[
        pl.BlockSpec((_RB, D), lambda i: (i, 0)),
        pl.BlockSpec((D, D), lambda i: (0, 0)),
        pl.BlockSpec((NCORES, _RB, D), lambda i: (0, i, 0)),
    ],
    out_specs=pl.BlockSpec((_RB, D), lambda i: (i, 0)),
    out_shape=jax.ShapeDtypeStruct((NPAD, D), jnp.float32),
)


def _tc_mid_body(s_ref, deg_ref, b_ref, w_ref, o_ref):
    dinv = _dinv(deg_ref)
    t = (s_ref[0] + s_ref[1]) * dinv + b_ref[...]
    r = jnp.maximum(t, 0.0)
    o_ref[...] = jnp.dot(r, w_ref[...], preferred_element_type=jnp.float32,
                         precision=lax.Precision.HIGHEST) * dinv


_tc_mid = pl.pallas_call(
    _tc_mid_body,
    grid=(_GRID,),
    in_specs=[
        pl.BlockSpec((NCORES, _RB, D), lambda i: (0, i, 0)),
        pl.BlockSpec((NCORES, _RB, D), lambda i: (0, i, 0)),
        pl.BlockSpec((1, D), lambda i: (0, 0)),
        pl.BlockSpec((D, D), lambda i: (0, 0)),
    ],
    out_specs=pl.BlockSpec((_RB, D), lambda i: (i, 0)),
    out_shape=jax.ShapeDtypeStruct((NPAD, D), jnp.float32),
)


def _tc_head_body(s_ref, deg_ref, b_ref, w_ref, b3_ref, o_ref):
    dinv = _dinv(deg_ref)
    t = (s_ref[0] + s_ref[1]) * dinv + b_ref[...]
    r = jnp.maximum(t, 0.0)
    lg = jnp.dot(r, w_ref[...], preferred_element_type=jnp.float32,
                 precision=lax.Precision.HIGHEST) + b3_ref[...]
    m = jnp.max(lg, axis=1, keepdims=True)
    e = jnp.exp(lg - m)
    o_ref[...] = e / jnp.sum(e, axis=1, keepdims=True)


_tc_head = pl.pallas_call(
    _tc_head_body,
    grid=(_GRID,),
    in_specs=[
        pl.BlockSpec((NCORES, _RB, D), lambda i: (0, i, 0)),
        pl.BlockSpec((NCORES, _RB, D), lambda i: (0, i, 0)),
        pl.BlockSpec((1, D), lambda i: (0, 0)),
        pl.BlockSpec((D, DOUT), lambda i: (0, 0)),
        pl.BlockSpec((1, DOUT), lambda i: (0, 0)),
    ],
    out_specs=pl.BlockSpec((_RB, DOUT), lambda i: (i, 0)),
    out_shape=jax.ShapeDtypeStruct((N, DOUT), jnp.float32),
)


def kernel(x, edge_index, W1, b1, W2, b2, W3, b3):
    src = edge_index[0].astype(jnp.int32)
    dst = edge_index[1].astype(jnp.int32)
    ppt = EPAD // NTILES - E // NTILES
    pad_rows = jnp.broadcast_to(N + jnp.arange(ppt, dtype=jnp.int32),
                                (NTILES, ppt))
    srcp = jnp.concatenate([src.reshape(NTILES, E // NTILES), pad_rows], axis=1)
    dstp = jnp.concatenate([dst.reshape(NTILES, E // NTILES), pad_rows], axis=1)
    srcp_m = srcp.reshape(NTILES, NBLK, BLKE)
    dstp_m = dstp.reshape(NTILES, NBLK, BLKE)
    dstp_d = dstp_m
    xp = jnp.pad(x, ((0, NPAD - N), (0, 0)))

    deg_p = _deg_kernel(dstp_d)
    v1 = _tc_in(xp, W1, deg_p)
    s1 = _msg_kernel(v1, srcp_m, dstp_m)
    v2 = _tc_mid(s1, deg_p, b1.reshape(1, D), W2)
    s2 = _msg_kernel(v2, srcp_m, dstp_m)
    return _tc_head(s2, deg_p, b2.reshape(1, D), W3, b3.reshape(1, DOUT))


# speedup vs baseline: 1.1326x; 1.1326x over previous
import functools

import jax
import jax.numpy as jnp
from jax import lax
from jax.experimental import pallas as pl
from jax.experimental.pallas import tpu as pltpu
from jax.experimental.pallas import tpu_sc as plsc

N = 10000
E = 320000
D = 128
DOUT = 64
NPAD = 10240
NCORES = 2
NSUB = 16
NTILES = NCORES * NSUB
NBLK = 80
BLKE = 128
MBLK = 160
MBLKE = 64
EPAD = NTILES * NBLK * BLKE
RPT = NPAD // NSUB

_RB = 1024
_GRID = NPAD // _RB

_mesh = plsc.VectorSubcoreMesh(core_axis_name="c", subcore_axis_name="s",
                               num_cores=NCORES, num_subcores=NSUB)


@functools.partial(
    pl.kernel,
    out_type=jax.ShapeDtypeStruct((NCORES, NPAD, D), jnp.float32),
    mesh=_mesh,
    scratch_types=[
        pltpu.VMEM((NBLK, BLKE), jnp.int32),
        pltpu.VMEM((BLKE, D), jnp.float32),
        pltpu.VMEM_SHARED((NPAD, D), jnp.float32),
    ],
)
def _deg_kernel(dst_hbm, out_hbm, dst_v, ones_v, acc_sh):
    c = lax.axis_index("c")
    s = lax.axis_index("s")
    wid = s * NCORES + c
    pltpu.sync_copy(dst_hbm.at[wid], dst_v)
    zrow = jnp.zeros((16,), jnp.float32)
    for r in range(BLKE):
        for q in range(D // 16):
            ones_v[r, pl.ds(q * 16, 16)] = zrow
    for k in range(RPT // BLKE):
        pltpu.sync_copy(ones_v, acc_sh.at[pl.ds(s * RPT + k * BLKE, BLKE)])
    orow = jnp.ones((16,), jnp.float32)
    for r in range(BLKE):
        for q in range(D // 16):
            ones_v[r, pl.ds(q * 16, 16)] = orow
    plsc.subcore_barrier()

    def blk(j, carry):
        pltpu.sync_copy(ones_v, acc_sh.at[dst_v.at[j]], add=True)
        return carry

    lax.fori_loop(0, NBLK, blk, 0)
    plsc.subcore_barrier()
    pltpu.sync_copy(acc_sh.at[pl.ds(s * RPT, RPT)],
                    out_hbm.at[c, pl.ds(s * RPT, RPT)])


@functools.partial(
    pl.kernel,
    out_type=jax.ShapeDtypeStruct((NCORES, NPAD, D), jnp.float32),
    mesh=_mesh,
    scratch_types=[
        pltpu.VMEM((MBLK // 2, MBLKE), jnp.int32),
        pltpu.VMEM((MBLK // 2, MBLKE), jnp.int32),
        pltpu.VMEM((MBLKE, D), jnp.float32),
        pltpu.VMEM((MBLKE, D), jnp.float32),
        pltpu.VMEM_SHARED((NPAD, D), jnp.float32),
        pltpu.SemaphoreType.DMA,
        pltpu.SemaphoreType.DMA,
    ],
)
def _msg_kernel(v_hbm, src_hbm, dst_hbm, out_hbm,
                src_v, dst_v, rows_a, rows_b, acc_sh, sem_a, sem_b):
    c = lax.axis_index("c")
    s = lax.axis_index("s")
    wid = s * NCORES + c
    half = MBLK // 2
    zrow = jnp.zeros((16,), jnp.float32)
    for r in range(MBLKE):
        for q in range(D // 16):
            rows_a[r, pl.ds(q * 16, 16)] = zrow
            rows_b[r, pl.ds(q * 16, 16)] = zrow
    for k in range(RPT // (2 * MBLKE)):
        pltpu.sync_copy(rows_a, acc_sh.at[pl.ds(s * RPT + 2 * k * MBLKE, MBLKE)])
        pltpu.sync_copy(rows_b, acc_sh.at[pl.ds(s * RPT + (2 * k + 1) * MBLKE, MBLKE)])
    plsc.subcore_barrier()

    for ph in range(2):
        pltpu.sync_copy(src_hbm.at[wid, pl.ds(ph * half, half)], src_v)
        pltpu.sync_copy(dst_hbm.at[wid, pl.ds(ph * half, half)], dst_v)
        pltpu.async_copy(v_hbm.at[src_v.at[0]], rows_a, sem_a)

        def blk(i, carry):
            j = 2 * i
            pltpu.async_copy(v_hbm.at[src_v.at[j + 1]], rows_b, sem_b)
            pltpu.make_async_copy(v_hbm.at[src_v.at[j]], rows_a, sem_a).wait()
            pltpu.sync_copy(rows_a, acc_sh.at[dst_v.at[j]], add=True)
            pltpu.async_copy(v_hbm.at[src_v.at[(j + 2) % half]], rows_a, sem_a)
            pltpu.make_async_copy(v_hbm.at[src_v.at[j + 1]], rows_b, sem_b).wait()
            pltpu.sync_copy(rows_b, acc_sh.at[dst_v.at[j + 1]], add=True)
            return carry

        lax.fori_loop(0, half // 2, blk, 0)
        pltpu.make_async_copy(v_hbm.at[src_v.at[0]], rows_a, sem_a).wait()
    plsc.subcore_barrier()
    pltpu.sync_copy(acc_sh.at[pl.ds(s * RPT, RPT)],
                    out_hbm.at[c, pl.ds(s * RPT, RPT)])


def _dinv(deg_ref):
    deg = (deg_ref[0] + deg_ref[1])[:, 0:1]
    return jnp.where(deg > 0.0, 1.0 / jnp.sqrt(jnp.maximum(deg, 1.0)), 0.0)


def _tc_in_body(x_ref, w_ref, deg_ref, o_ref):
    o_ref[...] = jnp.dot(x_ref[...], w_ref[...],
                         preferred_element_type=jnp.float32,
                         precision=lax.Precision.HIGHEST) * _dinv(deg_ref)


_tc_in = pl.pallas_call(
    _tc_in_body,
    grid=(_GRID,),
    in_specs=[
        pl.BlockSpec((_RB, D), lambda i: (i, 0)),
        pl.BlockSpec((D, D), lambda i: (0, 0)),
        pl.BlockSpec((NCORES, _RB, D), lambda i: (0, i, 0)),
    ],
    out_specs=pl.BlockSpec((_RB, D), lambda i: (i, 0)),
    out_shape=jax.ShapeDtypeStruct((NPAD, D), jnp.float32),
)


def _tc_mid_body(s_ref, deg_ref, b_ref, w_ref, o_ref):
    dinv = _dinv(deg_ref)
    t = (s_ref[0] + s_ref[1]) * dinv + b_ref[...]
    r = jnp.maximum(t, 0.0)
    o_ref[...] = jnp.dot(r, w_ref[...], preferred_element_type=jnp.float32,
                         precision=lax.Precision.HIGHEST) * dinv


_tc_mid = pl.pallas_call(
    _tc_mid_body,
    grid=(_GRID,),
    in_specs=[
        pl.BlockSpec((NCORES, _RB, D), lambda i: (0, i, 0)),
        pl.BlockSpec((NCORES, _RB, D), lambda i: (0, i, 0)),
        pl.BlockSpec((1, D), lambda i: (0, 0)),
        pl.BlockSpec((D, D), lambda i: (0, 0)),
    ],
    out_specs=pl.BlockSpec((_RB, D), lambda i: (i, 0)),
    out_shape=jax.ShapeDtypeStruct((NPAD, D), jnp.float32),
)


def _tc_head_body(s_ref, deg_ref, b_ref, w_ref, b3_ref, o_ref):
    dinv = _dinv(deg_ref)
    t = (s_ref[0] + s_ref[1]) * dinv + b_ref[...]
    r = jnp.maximum(t, 0.0)
    lg = jnp.dot(r, w_ref[...], preferred_element_type=jnp.float32,
                 precision=lax.Precision.HIGHEST) + b3_ref[...]
    m = jnp.max(lg, axis=1, keepdims=True)
    e = jnp.exp(lg - m)
    o_ref[...] = e / jnp.sum(e, axis=1, keepdims=True)


_tc_head = pl.pallas_call(
    _tc_head_body,
    grid=(_GRID,),
    in_specs=[
        pl.BlockSpec((NCORES, _RB, D), lambda i: (0, i, 0)),
        pl.BlockSpec((NCORES, _RB, D), lambda i: (0, i, 0)),
        pl.BlockSpec((1, D), lambda i: (0, 0)),
        pl.BlockSpec((D, DOUT), lambda i: (0, 0)),
        pl.BlockSpec((1, DOUT), lambda i: (0, 0)),
    ],
    out_specs=pl.BlockSpec((_RB, DOUT), lambda i: (i, 0)),
    out_shape=jax.ShapeDtypeStruct((N, DOUT), jnp.float32),
)


def kernel(x, edge_index, W1, b1, W2, b2, W3, b3):
    src = edge_index[0].astype(jnp.int32)
    dst = edge_index[1].astype(jnp.int32)
    ppt = EPAD // NTILES - E // NTILES
    pad_rows = jnp.broadcast_to(N + jnp.arange(ppt, dtype=jnp.int32),
                                (NTILES, ppt))
    srcp = jnp.concatenate([src.reshape(NTILES, E // NTILES), pad_rows], axis=1)
    dstp = jnp.concatenate([dst.reshape(NTILES, E // NTILES), pad_rows], axis=1)
    srcp_m = srcp.reshape(NTILES, MBLK, MBLKE)
    dstp_m = dstp.reshape(NTILES, MBLK, MBLKE)
    dstp_d = dstp.reshape(NTILES, NBLK, BLKE)
    xp = jnp.pad(x, ((0, NPAD - N), (0, 0)))

    deg_p = _deg_kernel(dstp_d)
    v1 = _tc_in(xp, W1, deg_p)
    s1 = _msg_kernel(v1, srcp_m, dstp_m)
    v2 = _tc_mid(s1, deg_p, b1.reshape(1, D), W2)
    s2 = _msg_kernel(v2, srcp_m, dstp_m)
    return _tc_head(s2, deg_p, b2.reshape(1, D), W3, b3.reshape(1, DOUT))


# speedup vs baseline: 1.2599x; 1.1124x over previous
import functools

import jax
import jax.numpy as jnp
from jax import lax
from jax.experimental import pallas as pl
from jax.experimental.pallas import tpu as pltpu
from jax.experimental.pallas import tpu_sc as plsc

N = 10000
E = 320000
D = 128
DOUT = 64
NPAD = 10240
NCORES = 2
NSUB = 16
NTILES = NCORES * NSUB
NBLK = 80
BLKE = 128
MBLK = 80
MBLKE = 128
EPAD = NTILES * NBLK * BLKE
RPT = NPAD // NSUB

_RB = 1024
_GRID = NPAD // _RB

_mesh = plsc.VectorSubcoreMesh(core_axis_name="c", subcore_axis_name="s",
                               num_cores=NCORES, num_subcores=NSUB)


@functools.partial(
    pl.kernel,
    out_type=jax.ShapeDtypeStruct((NCORES, NPAD, D), jnp.float32),
    mesh=_mesh,
    scratch_types=[
        pltpu.VMEM((NBLK, BLKE), jnp.int32),
        pltpu.VMEM((BLKE, D), jnp.float32),
        pltpu.VMEM_SHARED((NPAD, D), jnp.float32),
    ],
)
def _deg_kernel(dst_hbm, out_hbm, dst_v, ones_v, acc_sh):
    c = lax.axis_index("c")
    s = lax.axis_index("s")
    wid = s * NCORES + c
    pltpu.sync_copy(dst_hbm.at[wid], dst_v)
    zrow = jnp.zeros((16,), jnp.float32)
    for r in range(BLKE):
        for q in range(D // 16):
            ones_v[r, pl.ds(q * 16, 16)] = zrow
    for k in range(RPT // BLKE):
        pltpu.sync_copy(ones_v, acc_sh.at[pl.ds(s * RPT + k * BLKE, BLKE)])
    orow = jnp.ones((16,), jnp.float32)
    for r in range(BLKE):
        for q in range(D // 16):
            ones_v[r, pl.ds(q * 16, 16)] = orow
    plsc.subcore_barrier()

    def blk(j, carry):
        pltpu.sync_copy(ones_v, acc_sh.at[dst_v.at[j]], add=True)
        return carry

    lax.fori_loop(0, NBLK, blk, 0)
    plsc.subcore_barrier()
    pltpu.sync_copy(acc_sh.at[pl.ds(s * RPT, RPT)],
                    out_hbm.at[c, pl.ds(s * RPT, RPT)])


@functools.partial(
    pl.kernel,
    out_type=jax.ShapeDtypeStruct((NCORES, NPAD, D), jnp.float32),
    mesh=_mesh,
    scratch_types=[
        pltpu.VMEM((MBLK // 2, MBLKE), jnp.int32),
        pltpu.VMEM((MBLK // 2, MBLKE), jnp.int32),
        pltpu.VMEM((MBLKE, D), jnp.float32),
        pltpu.VMEM((MBLKE, D), jnp.float32),
        pltpu.VMEM_SHARED((NPAD, D), jnp.float32),
        pltpu.SemaphoreType.DMA,
        pltpu.SemaphoreType.DMA,
    ],
)
def _msg_kernel(v_hbm, src_hbm, dst_hbm, out_hbm,
                src_v, dst_v, rows_a, rows_b, acc_sh, sem_a, sem_b):
    c = lax.axis_index("c")
    s = lax.axis_index("s")
    wid = s * NCORES + c
    half = MBLK // 2
    zrow = jnp.zeros((16,), jnp.float32)
    for r in range(MBLKE):
        for q in range(D // 16):
            rows_a[r, pl.ds(q * 16, 16)] = zrow
            rows_b[r, pl.ds(q * 16, 16)] = zrow
    for k in range(RPT // (2 * MBLKE)):
        pltpu.sync_copy(rows_a, acc_sh.at[pl.ds(s * RPT + 2 * k * MBLKE, MBLKE)])
        pltpu.sync_copy(rows_b, acc_sh.at[pl.ds(s * RPT + (2 * k + 1) * MBLKE, MBLKE)])
    plsc.subcore_barrier()

    for ph in range(2):
        pltpu.sync_copy(src_hbm.at[wid, pl.ds(ph * half, half)], src_v)
        pltpu.sync_copy(dst_hbm.at[wid, pl.ds(ph * half, half)], dst_v)
        pltpu.async_copy(v_hbm.at[src_v.at[0]], rows_a, sem_a)

        def blk(i, carry):
            j = 2 * i
            pltpu.async_copy(v_hbm.at[src_v.at[j + 1]], rows_b, sem_b)
            pltpu.make_async_copy(v_hbm.at[src_v.at[j]], rows_a, sem_a).wait()
            pltpu.sync_copy(rows_a, acc_sh.at[dst_v.at[j]], add=True)
            pltpu.async_copy(v_hbm.at[src_v.at[(j + 2) % half]], rows_a, sem_a)
            pltpu.make_async_copy(v_hbm.at[src_v.at[j + 1]], rows_b, sem_b).wait()
            pltpu.sync_copy(rows_b, acc_sh.at[dst_v.at[j + 1]], add=True)
            return carry

        lax.fori_loop(0, half // 2, blk, 0)
        pltpu.make_async_copy(v_hbm.at[src_v.at[0]], rows_a, sem_a).wait()
    plsc.subcore_barrier()
    pltpu.sync_copy(acc_sh.at[pl.ds(s * RPT, RPT)],
                    out_hbm.at[c, pl.ds(s * RPT, RPT)])


def _dinv(deg_ref):
    deg = (deg_ref[0] + deg_ref[1])[:, 0:1]
    return jnp.where(deg > 0.0, 1.0 / jnp.sqrt(jnp.maximum(deg, 1.0)), 0.0)


def _tc_in_body(x_ref, w_ref, deg_ref, o_ref):
    o_ref[...] = jnp.dot(x_ref[...], w_ref[...],
                         preferred_element_type=jnp.float32,
                         precision=lax.Precision.HIGHEST) * _dinv(deg_ref)


_tc_in = pl.pallas_call(
    _tc_in_body,
    grid=(_GRID,),
    in_specs=[
        pl.BlockSpec((_RB, D), lambda i: (i, 0)),
        pl.BlockSpec((D, D), lambda i: (0, 0)),
        pl.BlockSpec((NCORES, _RB, D), lambda i: (0, i, 0)),
    ],
    out_specs=pl.BlockSpec((_RB, D), lambda i: (i, 0)),
    out_shape=jax.ShapeDtypeStruct((NPAD, D), jnp.float32),
)


def _tc_mid_body(s_ref, deg_ref, b_ref, w_ref, o_ref):
    dinv = _dinv(deg_ref)
    t = (s_ref[0] + s_ref[1]) * dinv + b_ref[...]
    r = jnp.maximum(t, 0.0)
    o_ref[...] = jnp.dot(r, w_ref[...], preferred_element_type=jnp.float32,
                         precision=lax.Precision.HIGHEST) * dinv


_tc_mid = pl.pallas_call(
    _tc_mid_body,
    grid=(_GRID,),
    in_specs=[
        pl.BlockSpec((NCORES, _RB, D), lambda i: (0, i, 0)),
        pl.BlockSpec((NCORES, _RB, D), lambda i: (0, i, 0)),
        pl.BlockSpec((1, D), lambda i: (0, 0)),
        pl.BlockSpec((D, D), lambda i: (0, 0)),
    ],
    out_specs=pl.BlockSpec((_RB, D), lambda i: (i, 0)),
    out_shape=jax.ShapeDtypeStruct((NPAD, D), jnp.float32),
)


def _tc_head_body(s_ref, deg_ref, b_ref, w_ref, b3_ref, o_ref):
    dinv = _dinv(deg_ref)
    t = (s_ref[0] + s_ref[1]) * dinv + b_ref[...]
    r = jnp.maximum(t, 0.0)
    lg = jnp.dot(r, w_ref[...], preferred_element_type=jnp.float32,
                 precision=lax.Precision.HIGHEST) + b3_ref[...]
    m = jnp.max(lg, axis=1, keepdims=True)
    e = jnp.exp(lg - m)
    o_ref[...] = e / jnp.sum(e, axis=1, keepdims=True)


_tc_head = pl.pallas_call(
    _tc_head_body,
    grid=(_GRID,),
    in_specs=[
        pl.BlockSpec((NCORES, _RB, D), lambda i: (0, i, 0)),
        pl.BlockSpec((NCORES, _RB, D), lambda i: (0, i, 0)),
        pl.BlockSpec((1, D), lambda i: (0, 0)),
        pl.BlockSpec((D, DOUT), lambda i: (0, 0)),
        pl.BlockSpec((1, DOUT), lambda i: (0, 0)),
    ],
    out_specs=pl.BlockSpec((_RB, DOUT), lambda i: (i, 0)),
    out_shape=jax.ShapeDtypeStruct((N, DOUT), jnp.float32),
)


def kernel(x, edge_index, W1, b1, W2, b2, W3, b3):
    src = edge_index[0].astype(jnp.int32)
    dst = edge_index[1].astype(jnp.int32)
    ppt = EPAD // NTILES - E // NTILES
    pad_rows = jnp.broadcast_to(N + jnp.arange(ppt, dtype=jnp.int32),
                                (NTILES, ppt))
    srcp = jnp.concatenate([src.reshape(NTILES, E // NTILES), pad_rows], axis=1)
    dstp = jnp.concatenate([dst.reshape(NTILES, E // NTILES), pad_rows], axis=1)
    srcp_m = srcp.reshape(NTILES, MBLK, MBLKE)
    dstp_m = dstp.reshape(NTILES, MBLK, MBLKE)
    dstp_d = dstp.reshape(NTILES, NBLK, BLKE)
    xp = jnp.pad(x, ((0, NPAD - N), (0, 0)))

    deg_p = _deg_kernel(dstp_d)
    v1 = _tc_in(xp, W1, deg_p)
    s1 = _msg_kernel(v1, srcp_m, dstp_m)
    v2 = _tc_mid(s1, deg_p, b1.reshape(1, D), W2)
    s2 = _msg_kernel(v2, srcp_m, dstp_m)
    return _tc_head(s2, deg_p, b2.reshape(1, D), W3, b3.reshape(1, DOUT))
